# Initial kernel scaffold; baseline (speedup 1.0000x reference)
import jax
import jax.numpy as jnp
from jax.experimental import pallas as pl


def kernel(seq1, seq2, seq3, seq4, adj, aug_adj1, aug_adj2, W_gcn, b_gcn, prelu_a, W_bil, b_bil):
    raise NotImplementedError("write your pallas kernel here")


# speedup vs baseline: 1.3073x; 1.3073x over previous
import functools

import jax
import jax.numpy as jnp
from jax.experimental import pallas as pl
from jax.experimental.pallas import tpu as pltpu


def _prelu(x, a):
    return jnp.where(x >= 0, x, a * x)


def _proj_body(seq1_ref, seq2_ref, w_ref, s_ref):
    w = w_ref[...].astype(jnp.bfloat16)
    dn = (((1,), (1,)), ((), ()))
    s1 = jax.lax.dot_general(seq1_ref[0].astype(jnp.bfloat16), w, dn,
                             preferred_element_type=jnp.float32)
    s2 = jax.lax.dot_general(seq2_ref[0].astype(jnp.bfloat16), w, dn,
                             preferred_element_type=jnp.float32)
    s_ref[...] = jnp.concatenate([s1, s2], axis=1).astype(jnp.bfloat16)


def _gcn_body(adj_ref, aug1_ref, aug2_ref, s_ref, b_ref, a_ref,
              h0_ref, h2_ref, m1_ref, m3_ref, *, h):
    a = a_ref[0, 0]
    b = b_ref[...]
    s = s_ref[...]
    adj = adj_ref[0].astype(jnp.bfloat16)
    acc = jnp.dot(adj, s, preferred_element_type=jnp.float32)
    h0_ref[...] = _prelu(acc[:, :h] + b, a)
    h2_ref[...] = _prelu(acc[:, h:] + b, a)
    s1 = s[:, :h]
    p1 = _prelu(jnp.dot(aug1_ref[0].astype(jnp.bfloat16), s1,
                        preferred_element_type=jnp.float32) + b, a)
    p3 = _prelu(jnp.dot(aug2_ref[0].astype(jnp.bfloat16), s1,
                        preferred_element_type=jnp.float32) + b, a)
    m1_ref[...] = jnp.sum(p1, axis=0, keepdims=True)[None]
    m3_ref[...] = jnp.sum(p3, axis=0, keepdims=True)[None]


def _score_body(h0_ref, h2_ref, m1_ref, m3_ref, wb_ref, bb_ref,
                o1_ref, o2_ref, *, n_nodes):
    m1 = jnp.sum(m1_ref[...], axis=0)
    m3 = jnp.sum(m3_ref[...], axis=0)
    inv_n = jnp.float32(1.0 / n_nodes)
    c = jax.nn.sigmoid(m1 * inv_n) + jax.nn.sigmoid(m3 * inv_n)
    wb = wb_ref[0]
    v = jax.lax.dot_general(c, wb, (((1,), (1,)), ((), ())),
                            preferred_element_type=jnp.float32)
    two_bb = 2.0 * bb_ref[0, 0]
    o1_ref[...] = jnp.sum(h0_ref[...] * v, axis=1, keepdims=True) + two_bb
    o2_ref[...] = jnp.sum(h2_ref[...] * v, axis=1, keepdims=True) + two_bb


def kernel(seq1, seq2, seq3, seq4, adj, aug_adj1, aug_adj2,
           W_gcn, b_gcn, prelu_a, W_bil, b_bil):
    del seq3, seq4
    _, n, n_in = seq1.shape
    h = W_gcn.shape[0]
    bm = 200 if n % 200 == 0 else (8 if n % 8 == 0 else 1)
    num_i = n // bm

    s = pl.pallas_call(
        _proj_body,
        in_specs=[
            pl.BlockSpec((1, n, n_in), lambda: (0, 0, 0)),
            pl.BlockSpec((1, n, n_in), lambda: (0, 0, 0)),
            pl.BlockSpec((h, n_in), lambda: (0, 0)),
        ],
        out_specs=pl.BlockSpec((n, 2 * h), lambda: (0, 0)),
        out_shape=jax.ShapeDtypeStruct((n, 2 * h), jnp.bfloat16),
    )(seq1, seq2, W_gcn)

    b2 = b_gcn.reshape(1, h)
    a2 = prelu_a.reshape(1, 1)
    h0, h2, m1p, m3p = pl.pallas_call(
        functools.partial(_gcn_body, h=h),
        grid=(num_i,),
        in_specs=[
            pl.BlockSpec((1, bm, n), lambda i: (0, i, 0)),
            pl.BlockSpec((1, bm, n), lambda i: (0, i, 0)),
            pl.BlockSpec((1, bm, n), lambda i: (0, i, 0)),
            pl.BlockSpec((n, 2 * h), lambda i: (0, 0)),
            pl.BlockSpec((1, h), lambda i: (0, 0)),
            pl.BlockSpec((1, 1), lambda i: (0, 0)),
        ],
        out_specs=[
            pl.BlockSpec((bm, h), lambda i: (i, 0)),
            pl.BlockSpec((bm, h), lambda i: (i, 0)),
            pl.BlockSpec((1, 1, h), lambda i: (i, 0, 0)),
            pl.BlockSpec((1, 1, h), lambda i: (i, 0, 0)),
        ],
        out_shape=[
            jax.ShapeDtypeStruct((n, h), jnp.float32),
            jax.ShapeDtypeStruct((n, h), jnp.float32),
            jax.ShapeDtypeStruct((num_i, 1, h), jnp.float32),
            jax.ShapeDtypeStruct((num_i, 1, h), jnp.float32),
        ],
        compiler_params=pltpu.CompilerParams(
            dimension_semantics=("parallel",)),
    )(adj, aug_adj1, aug_adj2, s, b2, a2)

    o1, o2 = pl.pallas_call(
        functools.partial(_score_body, n_nodes=float(n)),
        in_specs=[
            pl.BlockSpec((n, h), lambda: (0, 0)),
            pl.BlockSpec((n, h), lambda: (0, 0)),
            pl.BlockSpec((num_i, 1, h), lambda: (0, 0, 0)),
            pl.BlockSpec((num_i, 1, h), lambda: (0, 0, 0)),
            pl.BlockSpec((1, h, h), lambda: (0, 0, 0)),
            pl.BlockSpec((1, 1), lambda: (0, 0)),
        ],
        out_specs=[
            pl.BlockSpec((n, 1), lambda: (0, 0)),
            pl.BlockSpec((n, 1), lambda: (0, 0)),
        ],
        out_shape=[
            jax.ShapeDtypeStruct((n, 1), jnp.float32),
            jax.ShapeDtypeStruct((n, 1), jnp.float32),
        ],
    )(h0, h2, m1p, m3p, W_bil, b_bil.reshape(1, 1))

    return jnp.concatenate([o1.reshape(1, n), o2.reshape(1, n)], axis=1)


# speedup vs baseline: 1.3301x; 1.0175x over previous
import functools

import jax
import jax.numpy as jnp
from jax.experimental import pallas as pl
from jax.experimental.pallas import tpu as pltpu


def _prelu(x, a):
    return jnp.where(x >= 0, x, a * x)


_DN_T = (((1,), (1,)), ((), ()))


def _aug_body(aug1_ref, aug2_ref, seq1_ref, w_ref, b_ref, a_ref,
              m1_ref, m3_ref, s1_ref):
    i = pl.program_id(0)

    @pl.when(i == 0)
    def _init():
        w = w_ref[...].astype(jnp.bfloat16)
        s1_ref[...] = jax.lax.dot_general(
            seq1_ref[0].astype(jnp.bfloat16), w, _DN_T,
            preferred_element_type=jnp.float32).astype(jnp.bfloat16)
        m1_ref[...] = jnp.zeros_like(m1_ref)
        m3_ref[...] = jnp.zeros_like(m3_ref)

    a = a_ref[0, 0]
    b = b_ref[...]
    s1 = s1_ref[...]
    p1 = _prelu(jnp.dot(aug1_ref[0].astype(jnp.bfloat16), s1,
                        preferred_element_type=jnp.float32) + b, a)
    p3 = _prelu(jnp.dot(aug2_ref[0].astype(jnp.bfloat16), s1,
                        preferred_element_type=jnp.float32) + b, a)
    m1_ref[...] += jnp.sum(p1, axis=0, keepdims=True)
    m3_ref[...] += jnp.sum(p3, axis=0, keepdims=True)


def _adj_body(adj_ref, seq1_ref, seq2_ref, w_ref, b_ref, a_ref,
              m1_ref, m3_ref, wb_ref, bb_ref,
              o1_ref, o2_ref, s_ref, v_ref, *, h, inv_n):
    i = pl.program_id(0)

    @pl.when(i == 0)
    def _init():
        w = w_ref[...].astype(jnp.bfloat16)
        s_ref[:, :h] = jax.lax.dot_general(
            seq1_ref[0].astype(jnp.bfloat16), w, _DN_T,
            preferred_element_type=jnp.float32).astype(jnp.bfloat16)
        s_ref[:, h:] = jax.lax.dot_general(
            seq2_ref[0].astype(jnp.bfloat16), w, _DN_T,
            preferred_element_type=jnp.float32).astype(jnp.bfloat16)
        c = (jax.nn.sigmoid(m1_ref[...] * inv_n)
             + jax.nn.sigmoid(m3_ref[...] * inv_n))
        v_ref[...] = jax.lax.dot_general(
            c, wb_ref[0], _DN_T, preferred_element_type=jnp.float32)

    a = a_ref[0, 0]
    b = b_ref[...]
    s = s_ref[...]
    adj = adj_ref[0].astype(jnp.bfloat16)
    acc = jnp.dot(adj, s, preferred_element_type=jnp.float32)
    h0 = _prelu(acc[:, :h] + b, a)
    h2 = _prelu(acc[:, h:] + b, a)
    v = v_ref[...]
    two_bb = 2.0 * bb_ref[0, 0]
    o1_ref[...] = jnp.sum(h0 * v, axis=1, keepdims=True) + two_bb
    o2_ref[...] = jnp.sum(h2 * v, axis=1, keepdims=True) + two_bb


def kernel(seq1, seq2, seq3, seq4, adj, aug_adj1, aug_adj2,
           W_gcn, b_gcn, prelu_a, W_bil, b_bil):
    del seq3, seq4
    _, n, n_in = seq1.shape
    h = W_gcn.shape[0]
    bm = 200 if n % 200 == 0 else (8 if n % 8 == 0 else 1)
    num_i = n // bm

    b2 = b_gcn.reshape(1, h)
    a2 = prelu_a.reshape(1, 1)

    m1, m3 = pl.pallas_call(
        _aug_body,
        grid=(num_i,),
        in_specs=[
            pl.BlockSpec((1, bm, n), lambda i: (0, i, 0)),
            pl.BlockSpec((1, bm, n), lambda i: (0, i, 0)),
            pl.BlockSpec((1, n, n_in), lambda i: (0, 0, 0)),
            pl.BlockSpec((h, n_in), lambda i: (0, 0)),
            pl.BlockSpec((1, h), lambda i: (0, 0)),
            pl.BlockSpec((1, 1), lambda i: (0, 0)),
        ],
        out_specs=[
            pl.BlockSpec((1, h), lambda i: (0, 0)),
            pl.BlockSpec((1, h), lambda i: (0, 0)),
        ],
        out_shape=[
            jax.ShapeDtypeStruct((1, h), jnp.float32),
            jax.ShapeDtypeStruct((1, h), jnp.float32),
        ],
        scratch_shapes=[pltpu.VMEM((n, h), jnp.bfloat16)],
    )(aug_adj1, aug_adj2, seq1, W_gcn, b2, a2)

    o1, o2 = pl.pallas_call(
        functools.partial(_adj_body, h=h, inv_n=float(1.0 / n)),
        grid=(num_i,),
        in_specs=[
            pl.BlockSpec((1, bm, n), lambda i: (0, i, 0)),
            pl.BlockSpec((1, n, n_in), lambda i: (0, 0, 0)),
            pl.BlockSpec((1, n, n_in), lambda i: (0, 0, 0)),
            pl.BlockSpec((h, n_in), lambda i: (0, 0)),
            pl.BlockSpec((1, h), lambda i: (0, 0)),
            pl.BlockSpec((1, 1), lambda i: (0, 0)),
            pl.BlockSpec((1, h), lambda i: (0, 0)),
            pl.BlockSpec((1, h), lambda i: (0, 0)),
            pl.BlockSpec((1, h, h), lambda i: (0, 0, 0)),
            pl.BlockSpec((1, 1), lambda i: (0, 0)),
        ],
        out_specs=[
            pl.BlockSpec((bm, 1), lambda i: (i, 0)),
            pl.BlockSpec((bm, 1), lambda i: (i, 0)),
        ],
        out_shape=[
            jax.ShapeDtypeStruct((n, 1), jnp.float32),
            jax.ShapeDtypeStruct((n, 1), jnp.float32),
        ],
        scratch_shapes=[
            pltpu.VMEM((n, 2 * h), jnp.bfloat16),
            pltpu.VMEM((1, h), jnp.float32),
        ],
    )(adj, seq1, seq2, W_gcn, b2, a2, m1, m3, W_bil, b_bil.reshape(1, 1))

    return jnp.concatenate([o1.reshape(1, n), o2.reshape(1, n)], axis=1)


# speedup vs baseline: 1.3474x; 1.0130x over previous
import functools

import jax
import jax.numpy as jnp
from jax.experimental import pallas as pl
from jax.experimental.pallas import tpu as pltpu


def _prelu(x, a):
    return jnp.where(x >= 0, x, a * x)


_DN_T = (((1,), (1,)), ((), ()))


def _aug_body(aug1_ref, aug2_ref, seq1_ref, seq2_ref, w_ref, b_ref, a_ref,
              s_ref, m1_ref, m3_ref, *, h):
    i = pl.program_id(0)

    @pl.when(i == 0)
    def _init():
        w = w_ref[...].astype(jnp.bfloat16)
        s_ref[:, :h] = jax.lax.dot_general(
            seq1_ref[0].astype(jnp.bfloat16), w, _DN_T,
            preferred_element_type=jnp.float32).astype(jnp.bfloat16)
        s_ref[:, h:] = jax.lax.dot_general(
            seq2_ref[0].astype(jnp.bfloat16), w, _DN_T,
            preferred_element_type=jnp.float32).astype(jnp.bfloat16)
        m1_ref[...] = jnp.zeros_like(m1_ref)
        m3_ref[...] = jnp.zeros_like(m3_ref)

    a = a_ref[0, 0]
    b = b_ref[...]
    s1 = s_ref[:, :h]
    p1 = _prelu(jnp.dot(aug1_ref[0].astype(jnp.bfloat16), s1,
                        preferred_element_type=jnp.float32) + b, a)
    p3 = _prelu(jnp.dot(aug2_ref[0].astype(jnp.bfloat16), s1,
                        preferred_element_type=jnp.float32) + b, a)
    m1_ref[...] += jnp.sum(p1, axis=0, keepdims=True)
    m3_ref[...] += jnp.sum(p3, axis=0, keepdims=True)


def _adj_body(adj_ref, s_ref, b_ref, a_ref, m1_ref, m3_ref, wb_ref, bb_ref,
              o1_ref, o2_ref, v_ref, *, h, inv_n):
    i = pl.program_id(0)

    @pl.when(i == 0)
    def _init():
        c = (jax.nn.sigmoid(m1_ref[...] * inv_n)
             + jax.nn.sigmoid(m3_ref[...] * inv_n))
        v_ref[...] = jax.lax.dot_general(
            c, wb_ref[0], _DN_T, preferred_element_type=jnp.float32)

    a = a_ref[0, 0]
    b = b_ref[...]
    s = s_ref[...]
    adj = adj_ref[0].astype(jnp.bfloat16)
    acc = jnp.dot(adj, s, preferred_element_type=jnp.float32)
    h0 = _prelu(acc[:, :h] + b, a)
    h2 = _prelu(acc[:, h:] + b, a)
    v = v_ref[...]
    two_bb = 2.0 * bb_ref[0, 0]
    o1_ref[...] = jnp.sum(h0 * v, axis=1, keepdims=True) + two_bb
    o2_ref[...] = jnp.sum(h2 * v, axis=1, keepdims=True) + two_bb


def kernel(seq1, seq2, seq3, seq4, adj, aug_adj1, aug_adj2,
           W_gcn, b_gcn, prelu_a, W_bil, b_bil):
    del seq3, seq4
    _, n, n_in = seq1.shape
    h = W_gcn.shape[0]
    bm1 = 200 if n % 200 == 0 else (8 if n % 8 == 0 else 1)
    bm2 = 400 if n % 400 == 0 else bm1

    b2 = b_gcn.reshape(1, h)
    a2 = prelu_a.reshape(1, 1)

    s, m1, m3 = pl.pallas_call(
        functools.partial(_aug_body, h=h),
        grid=(n // bm1,),
        in_specs=[
            pl.BlockSpec((1, bm1, n), lambda i: (0, i, 0)),
            pl.BlockSpec((1, bm1, n), lambda i: (0, i, 0)),
            pl.BlockSpec((1, n, n_in), lambda i: (0, 0, 0)),
            pl.BlockSpec((1, n, n_in), lambda i: (0, 0, 0)),
            pl.BlockSpec((h, n_in), lambda i: (0, 0)),
            pl.BlockSpec((1, h), lambda i: (0, 0)),
            pl.BlockSpec((1, 1), lambda i: (0, 0)),
        ],
        out_specs=[
            pl.BlockSpec((n, 2 * h), lambda i: (0, 0)),
            pl.BlockSpec((1, h), lambda i: (0, 0)),
            pl.BlockSpec((1, h), lambda i: (0, 0)),
        ],
        out_shape=[
            jax.ShapeDtypeStruct((n, 2 * h), jnp.bfloat16),
            jax.ShapeDtypeStruct((1, h), jnp.float32),
            jax.ShapeDtypeStruct((1, h), jnp.float32),
        ],
    )(aug_adj1, aug_adj2, seq1, seq2, W_gcn, b2, a2)

    o1, o2 = pl.pallas_call(
        functools.partial(_adj_body, h=h, inv_n=float(1.0 / n)),
        grid=(n // bm2,),
        in_specs=[
            pl.BlockSpec((1, bm2, n), lambda i: (0, i, 0)),
            pl.BlockSpec((n, 2 * h), lambda i: (0, 0)),
            pl.BlockSpec((1, h), lambda i: (0, 0)),
            pl.BlockSpec((1, 1), lambda i: (0, 0)),
            pl.BlockSpec((1, h), lambda i: (0, 0)),
            pl.BlockSpec((1, h), lambda i: (0, 0)),
            pl.BlockSpec((1, h, h), lambda i: (0, 0, 0)),
            pl.BlockSpec((1, 1), lambda i: (0, 0)),
        ],
        out_specs=[
            pl.BlockSpec((bm2, 1), lambda i: (i, 0)),
            pl.BlockSpec((bm2, 1), lambda i: (i, 0)),
        ],
        out_shape=[
            jax.ShapeDtypeStruct((n, 1), jnp.float32),
            jax.ShapeDtypeStruct((n, 1), jnp.float32),
        ],
        scratch_shapes=[pltpu.VMEM((1, h), jnp.float32)],
    )(adj, s, b2, a2, m1, m3, W_bil, b_bil.reshape(1, 1))

    return jnp.concatenate([o1.reshape(1, n), o2.reshape(1, n)], axis=1)
